# baseline (speedup 1.0000x reference)
import jax
import jax.numpy as jnp
from jax.experimental import pallas as pl
from jax.experimental.pallas import tpu as pltpu

N, D_IN, D_OUT, TOPK = 8192, 1024, 1024, 8
BLOCK_ROWS = 1024
LANES = 128
CHUNKS = D_OUT // LANES

_SORT_PAIRS = [(0, 1), (2, 3), (4, 5), (6, 7),
               (0, 2), (1, 3), (4, 6), (5, 7),
               (1, 2), (5, 6),
               (0, 4), (1, 5), (2, 6), (3, 7),
               (2, 4), (3, 5),
               (1, 2), (3, 4), (5, 6)]


def _row_topk_threshold(logits):
    c = [logits[:, j * LANES:(j + 1) * LANES] for j in range(CHUNKS)]
    for (i, j) in _SORT_PAIRS:
        hi = jnp.maximum(c[i], c[j])
        lo = jnp.minimum(c[i], c[j])
        c[i], c[j] = hi, lo
    for r in range(TOPK - 1):
        m = jnp.max(c[0], axis=1, keepdims=True)
        is_m = c[0] >= m
        for i in range(CHUNKS - 1 - r):
            c[i] = jnp.where(is_m, c[i + 1], c[i])
    return jnp.max(c[0], axis=1, keepdims=True)


def _body(x_ref, wr_ref, br_ref, w_ref, b_ref, o_ref, wrt_ref, wt_ref):
    @pl.when(pl.program_id(0) == 0)
    def _prep():
        wrt_ref[...] = wr_ref[...].T.astype(jnp.bfloat16)
        wt_ref[...] = w_ref[...].T.astype(jnp.bfloat16)

    x = x_ref[...].astype(jnp.bfloat16)
    logits = jax.lax.dot_general(
        x, wrt_ref[...], (((1,), (0,)), ((), ())),
        preferred_element_type=jnp.float32,
    ) + br_ref[...]
    orig = jax.lax.dot_general(
        x, wt_ref[...], (((1,), (0,)), ((), ())),
        preferred_element_type=jnp.float32,
    ) + b_ref[...]
    thresh = _row_topk_threshold(logits)
    o_ref[...] = jnp.where(logits >= thresh, orig, 0.0)


@jax.jit
def kernel(x, Wr, br, W, b):
    br2 = br.reshape(1, D_OUT)
    b2 = b.reshape(1, D_OUT)
    grid = (N // BLOCK_ROWS,)
    return pl.pallas_call(
        _body,
        grid=grid,
        in_specs=[
            pl.BlockSpec((BLOCK_ROWS, D_IN), lambda i: (i, 0)),
            pl.BlockSpec((D_OUT, D_IN), lambda i: (0, 0)),
            pl.BlockSpec((1, D_OUT), lambda i: (0, 0)),
            pl.BlockSpec((D_OUT, D_IN), lambda i: (0, 0)),
            pl.BlockSpec((1, D_OUT), lambda i: (0, 0)),
        ],
        out_specs=pl.BlockSpec((BLOCK_ROWS, D_OUT), lambda i: (i, 0)),
        out_shape=jax.ShapeDtypeStruct((N, D_OUT), jnp.float32),
        compiler_params=pltpu.CompilerParams(
            dimension_semantics=("parallel",),
        ),
        scratch_shapes=[
            pltpu.VMEM((D_IN, D_OUT), jnp.bfloat16),
            pltpu.VMEM((D_IN, D_OUT), jnp.bfloat16),
        ],
    )(x, Wr, br2, W, b2)


# speedup vs baseline: 1.0427x; 1.0427x over previous
import jax
import jax.numpy as jnp
from jax.experimental import pallas as pl
from jax.experimental.pallas import tpu as pltpu

N, D_IN, D_OUT, TOPK = 8192, 1024, 1024, 8
BLOCK_ROWS = 512
LANES = 128
CHUNKS = D_OUT // LANES

_SORT_PAIRS = [(0, 1), (2, 3), (4, 5), (6, 7),
               (0, 2), (1, 3), (4, 6), (5, 7),
               (1, 2), (5, 6),
               (0, 4), (1, 5), (2, 6), (3, 7),
               (2, 4), (3, 5),
               (1, 2), (3, 4), (5, 6)]


def _row_topk_threshold(logits):
    c = [logits[:, j * LANES:(j + 1) * LANES] for j in range(CHUNKS)]
    for (i, j) in _SORT_PAIRS:
        hi = jnp.maximum(c[i], c[j])
        lo = jnp.minimum(c[i], c[j])
        c[i], c[j] = hi, lo
    for r in range(TOPK - 1):
        m = jnp.max(c[0], axis=1, keepdims=True)
        is_m = c[0] >= m
        for i in range(CHUNKS - 1 - r):
            c[i] = jnp.where(is_m, c[i + 1], c[i])
    return jnp.max(c[0], axis=1, keepdims=True)


def _body(x_ref, wr_ref, br_ref, w_ref, b_ref, o_ref, wrt_ref, wt_ref):
    @pl.when(pl.program_id(0) == 0)
    def _prep():
        wrt_ref[...] = wr_ref[...].T.astype(jnp.bfloat16)
        wt_ref[...] = w_ref[...].T.astype(jnp.bfloat16)

    x = x_ref[...].astype(jnp.bfloat16)
    logits = jax.lax.dot_general(
        x, wrt_ref[...], (((1,), (0,)), ((), ())),
        preferred_element_type=jnp.float32,
    ) + br_ref[...]
    orig = jax.lax.dot_general(
        x, wt_ref[...], (((1,), (0,)), ((), ())),
        preferred_element_type=jnp.float32,
    ) + b_ref[...]
    thresh = _row_topk_threshold(logits)
    o_ref[...] = jnp.where(logits >= thresh, orig, 0.0)


@jax.jit
def kernel(x, Wr, br, W, b):
    br2 = br.reshape(1, D_OUT)
    b2 = b.reshape(1, D_OUT)
    grid = (N // BLOCK_ROWS,)
    return pl.pallas_call(
        _body,
        grid=grid,
        in_specs=[
            pl.BlockSpec((BLOCK_ROWS, D_IN), lambda i: (i, 0)),
            pl.BlockSpec((D_OUT, D_IN), lambda i: (0, 0)),
            pl.BlockSpec((1, D_OUT), lambda i: (0, 0)),
            pl.BlockSpec((D_OUT, D_IN), lambda i: (0, 0)),
            pl.BlockSpec((1, D_OUT), lambda i: (0, 0)),
        ],
        out_specs=pl.BlockSpec((BLOCK_ROWS, D_OUT), lambda i: (i, 0)),
        out_shape=jax.ShapeDtypeStruct((N, D_OUT), jnp.float32),
        compiler_params=pltpu.CompilerParams(
            dimension_semantics=("parallel",),
        ),
        scratch_shapes=[
            pltpu.VMEM((D_IN, D_OUT), jnp.bfloat16),
            pltpu.VMEM((D_IN, D_OUT), jnp.bfloat16),
        ],
    )(x, Wr, br2, W, b2)
